# baseline (speedup 1.0000x reference)
import functools

import jax
import jax.numpy as jnp
from jax import lax
from jax.experimental import pallas as pl
from jax.experimental.pallas import tpu as pltpu

B = 64
T = 72
N = 5
F = 10
HG = 64
H = 320
HA = 384
GP = 384
E_PAD = 32
N_VALID = 21
C = 8
NC = T // C
R = B * C


def _fused_kernel(ed_ref, xt_ref, wg_ref, bg_ref, wih_ref, whh_ref,
                  gout_ref, hout_ref, k_scr, h_scr, p_scr, gi_scr):
    c = pl.program_id(0)

    @pl.when(c == 0)
    def _init():
        s_row = ed_ref[0:1, :]
        d_row = ed_ref[1:2, :]
        n_iota = lax.broadcasted_iota(jnp.int32, (8, E_PAD), 0)
        e_iota = lax.broadcasted_iota(jnp.int32, (8, E_PAD), 1)
        valid = (e_iota < N_VALID).astype(jnp.float32)
        oh_s = (jnp.broadcast_to(s_row, (8, E_PAD)) == n_iota)
        oh_d = (jnp.broadcast_to(d_row, (8, E_PAD)) == n_iota)
        oh_s = oh_s.astype(jnp.float32) * valid
        oh_d = oh_d.astype(jnp.float32) * valid
        deg = jnp.sum(oh_d, axis=1, keepdims=True)
        dis = jnp.where(deg > 0, lax.rsqrt(deg), 0.0)
        dis_s = jnp.sum(oh_s * dis, axis=0, keepdims=True)
        dis_d = jnp.sum(oh_d * dis, axis=0, keepdims=True)
        norm = dis_s * dis_d
        a8 = lax.dot_general(oh_d * norm, oh_s, (((1,), (1,)), ((), ())),
                             preferred_element_type=jnp.float32)

        r_i = lax.broadcasted_iota(jnp.int32, (N * F, 8), 0)
        c8_i = lax.broadcasted_iota(jnp.int32, (N * F, 8), 1)
        e_r = ((r_i // F) == c8_i).astype(jnp.float32)
        a_sel = lax.dot_general(e_r, a8, (((1,), (1,)), ((), ())),
                                preferred_element_type=jnp.float32)
        n8_i = lax.broadcasted_iota(jnp.int32, (8, H), 0)
        cH_i = lax.broadcasted_iota(jnp.int32, (8, H), 1)
        e_c = ((cH_i // HG) == n8_i).astype(jnp.float32)
        a_exp = jnp.dot(a_sel, e_c,
                        preferred_element_type=jnp.float32)
        rf_i = lax.broadcasted_iota(jnp.int32, (N * F, F), 0)
        cf_i = lax.broadcasted_iota(jnp.int32, (N * F, F), 1)
        f_r = ((rf_i % F) == cf_i).astype(jnp.float32)
        w_mid = jnp.dot(f_r, wg_ref[:],
                        preferred_element_type=jnp.float32)
        h_i = lax.broadcasted_iota(jnp.int32, (HG, H), 0)
        ch_i = lax.broadcasted_iota(jnp.int32, (HG, H), 1)
        f_c = ((ch_i % HG) == h_i).astype(jnp.float32)
        w_exp = jnp.dot(w_mid, f_c,
                        preferred_element_type=jnp.float32)
        kc_i = lax.broadcasted_iota(jnp.int32, (N * F, HA - H), 1)
        k_scr[:] = jnp.concatenate(
            [a_exp * w_exp, jnp.zeros_like(kc_i, jnp.float32)], axis=1)

        hc_i = lax.broadcasted_iota(jnp.int32, (B, HA), 1)
        h_scr[:] = (hc_i == H).astype(jnp.float32)

        pr_i = lax.broadcasted_iota(jnp.int32, (R, R), 0)
        pc_i = lax.broadcasted_iota(jnp.int32, (R, R), 1)
        p_scr[:] = ((pr_i % B) * C + pr_i // B == pc_i).astype(jnp.bfloat16)

    xt = xt_ref[:].reshape(R, N * F)
    g = jnp.tanh(jnp.dot(xt.astype(jnp.bfloat16),
                         k_scr[:].astype(jnp.bfloat16),
                         preferred_element_type=jnp.float32)
                 + bg_ref[:])
    gout_ref[:] = g[:, 0:H].reshape(B, C, H)
    g16 = g.astype(jnp.bfloat16)

    h_aug = h_scr[:]
    ones_tail = h_aug[:, H:HA]
    hh = h_aug
    for i in range(C):
        gi_i = gi_scr[i * B:(i + 1) * B, :]
        gh = jnp.dot(hh.astype(jnp.bfloat16), whh_ref[:],
                     preferred_element_type=jnp.float32)
        t_rz = jnp.tanh(gi_i[:, 0:GP + H] + gh[:, 0:GP + H])
        r = 0.5 + 0.5 * t_rz[:, 0:H]
        z = 0.5 + 0.5 * t_rz[:, GP:GP + H]
        n = jnp.tanh(gi_i[:, 2 * GP:2 * GP + H] + r * gh[:, 2 * GP:2 * GP + H])
        h_new = n + z * (hh[:, 0:H] - n)
        hh = jnp.concatenate([h_new, ones_tail], axis=1)

        g_t_i = jnp.dot(p_scr[i * B:(i + 1) * B, :], g16,
                        preferred_element_type=jnp.float32
                        ).astype(jnp.bfloat16)
        gi_scr[i * B:(i + 1) * B, :] = jnp.dot(
            g_t_i, wih_ref[:], preferred_element_type=jnp.float32)

    h_next = jnp.where(c > 0, hh, h_aug)
    h_scr[:] = h_next
    hout_ref[:] = h_next[:, 0:H]


def _prep_gate_weights(W_ih, W_hh, b_ih, b_hh):
    zw = jnp.zeros((H, GP - H), jnp.float32)
    zr = jnp.zeros((HA - H - 1, 3 * GP), jnp.float32)

    def gate(w, gidx):
        blk = w[gidx * H:(gidx + 1) * H].T
        return blk if gidx == 2 else 0.5 * blk

    wih = jnp.concatenate(
        [gate(W_ih, 0), zw, gate(W_ih, 1), zw, gate(W_ih, 2), zw], axis=1)
    whh = jnp.concatenate(
        [gate(W_hh, 0), zw, gate(W_hh, 1), zw, gate(W_hh, 2), zw], axis=1)
    zb = jnp.zeros((GP - H,), jnp.float32)
    bih_row = jnp.concatenate(
        [0.5 * (b_ih[0:H] + b_hh[0:H]), zb,
         0.5 * (b_ih[H:2 * H] + b_hh[H:2 * H]), zb,
         b_ih[2 * H:3 * H], zb]).reshape(1, 3 * GP)
    bhh_row = jnp.concatenate(
        [jnp.zeros((2 * GP,), jnp.float32), b_hh[2 * H:3 * H], zb]
    ).reshape(1, 3 * GP)
    wih_aug = jnp.concatenate([wih, bih_row, zr], axis=0).astype(jnp.bfloat16)
    whh_aug = jnp.concatenate([whh, bhh_row, zr], axis=0).astype(jnp.bfloat16)
    return wih_aug, whh_aug


@functools.partial(jax.jit, static_argnames=())
def kernel(x, edge_index, W_gcn, b_gcn, W_ih, W_hh, b_ih, b_hh):
    loops = jnp.arange(N, dtype=edge_index.dtype)
    epad = jnp.zeros((2, E_PAD - N_VALID), jnp.int32)
    ed2 = jnp.concatenate(
        [edge_index.astype(jnp.int32), jnp.stack([loops, loops]), epad],
        axis=1)
    ed = jnp.concatenate([ed2, jnp.zeros((6, E_PAD), jnp.int32)], axis=0)

    xt = x.reshape(B, T, N * F)
    bg_aug = jnp.concatenate(
        [jnp.tile(b_gcn, N), jnp.full((1,), 20.0, jnp.float32),
         jnp.zeros((HA - H - 1,), jnp.float32)]).reshape(1, HA)
    wih_aug, whh_aug = _prep_gate_weights(W_ih, W_hh, b_ih, b_hh)

    clamp = lambda c: jnp.minimum(c, NC - 1)

    gout, h_last = pl.pallas_call(
        _fused_kernel,
        grid=(NC + 1,),
        in_specs=[
            pl.BlockSpec((8, E_PAD), lambda c: (0, 0)),
            pl.BlockSpec((B, C, N * F), lambda c: (0, clamp(c), 0)),
            pl.BlockSpec((F, HG), lambda c: (0, 0)),
            pl.BlockSpec((1, HA), lambda c: (0, 0)),
            pl.BlockSpec((HA, 3 * GP), lambda c: (0, 0)),
            pl.BlockSpec((HA, 3 * GP), lambda c: (0, 0)),
        ],
        out_specs=[
            pl.BlockSpec((B, C, H), lambda c: (0, clamp(c), 0)),
            pl.BlockSpec((B, H), lambda c: (0, 0)),
        ],
        out_shape=[
            jax.ShapeDtypeStruct((B, T, H), jnp.float32),
            jax.ShapeDtypeStruct((B, H), jnp.float32),
        ],
        scratch_shapes=[
            pltpu.VMEM((N * F, HA), jnp.float32),
            pltpu.VMEM((B, HA), jnp.float32),
            pltpu.VMEM((R, R), jnp.bfloat16),
            pltpu.VMEM((R, 3 * GP), jnp.float32),
        ],
        compiler_params=pltpu.CompilerParams(
            dimension_semantics=("arbitrary",),
        ),
    )(ed, xt, W_gcn, bg_aug, wih_aug, whh_aug)

    gcn_features = gout.reshape(B, T, N, HG)
    return gcn_features, h_last


# speedup vs baseline: 1.0365x; 1.0365x over previous
import functools

import jax
import jax.numpy as jnp
from jax import lax
from jax.experimental import pallas as pl
from jax.experimental.pallas import tpu as pltpu

B = 64
T = 72
N = 5
F = 10
HG = 64
H = 320
HA = 384
GW = 3 * H
E_PAD = 32
N_VALID = 21
C = 8
NC = T // C
R = B * C


def _fused_kernel(ed_ref, xt_ref, wg_ref, bg_ref, wih_ref, whh_ref,
                  gout_ref, hout_ref, k_scr, h_scr, p_scr):
    c = pl.program_id(0)

    @pl.when(c == 0)
    def _init():
        s_row = ed_ref[0:1, :]
        d_row = ed_ref[1:2, :]
        n_iota = lax.broadcasted_iota(jnp.int32, (8, E_PAD), 0)
        e_iota = lax.broadcasted_iota(jnp.int32, (8, E_PAD), 1)
        valid = (e_iota < N_VALID).astype(jnp.float32)
        oh_s = (jnp.broadcast_to(s_row, (8, E_PAD)) == n_iota)
        oh_d = (jnp.broadcast_to(d_row, (8, E_PAD)) == n_iota)
        oh_s = oh_s.astype(jnp.float32) * valid
        oh_d = oh_d.astype(jnp.float32) * valid
        deg = jnp.sum(oh_d, axis=1, keepdims=True)
        dis = jnp.where(deg > 0, lax.rsqrt(deg), 0.0)
        dis_s = jnp.sum(oh_s * dis, axis=0, keepdims=True)
        dis_d = jnp.sum(oh_d * dis, axis=0, keepdims=True)
        norm = dis_s * dis_d
        a8 = lax.dot_general(oh_d * norm, oh_s, (((1,), (1,)), ((), ())),
                             preferred_element_type=jnp.float32)

        r_i = lax.broadcasted_iota(jnp.int32, (N * F, 8), 0)
        c8_i = lax.broadcasted_iota(jnp.int32, (N * F, 8), 1)
        e_r = ((r_i // F) == c8_i).astype(jnp.float32)
        a_sel = lax.dot_general(e_r, a8, (((1,), (1,)), ((), ())),
                                preferred_element_type=jnp.float32)
        n8_i = lax.broadcasted_iota(jnp.int32, (8, H), 0)
        cH_i = lax.broadcasted_iota(jnp.int32, (8, H), 1)
        e_c = ((cH_i // HG) == n8_i).astype(jnp.float32)
        a_exp = jnp.dot(a_sel, e_c,
                        preferred_element_type=jnp.float32)
        rf_i = lax.broadcasted_iota(jnp.int32, (N * F, F), 0)
        cf_i = lax.broadcasted_iota(jnp.int32, (N * F, F), 1)
        f_r = ((rf_i % F) == cf_i).astype(jnp.float32)
        w_mid = jnp.dot(f_r, wg_ref[:],
                        preferred_element_type=jnp.float32)
        h_i = lax.broadcasted_iota(jnp.int32, (HG, H), 0)
        ch_i = lax.broadcasted_iota(jnp.int32, (HG, H), 1)
        f_c = ((ch_i % HG) == h_i).astype(jnp.float32)
        w_exp = jnp.dot(w_mid, f_c,
                        preferred_element_type=jnp.float32)
        kc_i = lax.broadcasted_iota(jnp.int32, (N * F, HA - H), 1)
        k_scr[:] = (jnp.concatenate(
            [a_exp * w_exp, jnp.zeros_like(kc_i, jnp.float32)],
            axis=1)).astype(jnp.bfloat16)

        hc_i = lax.broadcasted_iota(jnp.int32, (B, HA), 1)
        h_scr[:] = (hc_i == H).astype(jnp.float32)

        pr_i = lax.broadcasted_iota(jnp.int32, (R, R), 0)
        pc_i = lax.broadcasted_iota(jnp.int32, (R, R), 1)
        p_scr[:] = ((pr_i % B) * C + pr_i // B == pc_i).astype(jnp.bfloat16)

    xt = xt_ref[:].reshape(R, N * F)
    g = jnp.tanh(jnp.dot(xt.astype(jnp.bfloat16), k_scr[:],
                         preferred_element_type=jnp.float32)
                 + bg_ref[:])
    gout_ref[:] = g[:, 0:H].reshape(B, C, H)

    g_t = jnp.dot(p_scr[:], g.astype(jnp.bfloat16),
                  preferred_element_type=jnp.float32
                  ).astype(jnp.bfloat16)

    gi = jnp.dot(g_t, wih_ref[:],
                 preferred_element_type=jnp.float32)

    h_aug = h_scr[:]
    ones_tail = h_aug[:, H:HA]
    hh = h_aug
    for i in range(C):
        gi_i = gi[i * B:(i + 1) * B, :]
        gh = jnp.dot(hh.astype(jnp.bfloat16), whh_ref[:],
                     preferred_element_type=jnp.float32)
        t_rz = jnp.tanh(gi_i[:, 0:2 * H] + gh[:, 0:2 * H])
        r = 0.5 + 0.5 * t_rz[:, 0:H]
        z = 0.5 + 0.5 * t_rz[:, H:2 * H]
        n = jnp.tanh(gi_i[:, 2 * H:3 * H] + r * gh[:, 2 * H:3 * H])
        h_new = n + z * (hh[:, 0:H] - n)
        hh = jnp.concatenate([h_new, ones_tail], axis=1)
    h_scr[:] = hh
    hout_ref[:] = hh[:, 0:H]


def _prep_gate_weights(W_ih, W_hh, b_ih, b_hh):
    zr = jnp.zeros((HA - H - 1, GW), jnp.float32)

    def gate(w, gidx):
        blk = w[gidx * H:(gidx + 1) * H].T
        return blk if gidx == 2 else 0.5 * blk

    wih = jnp.concatenate(
        [gate(W_ih, 0), gate(W_ih, 1), gate(W_ih, 2)], axis=1)
    whh = jnp.concatenate(
        [gate(W_hh, 0), gate(W_hh, 1), gate(W_hh, 2)], axis=1)
    bih_row = jnp.concatenate(
        [0.5 * (b_ih[0:H] + b_hh[0:H]),
         0.5 * (b_ih[H:2 * H] + b_hh[H:2 * H]),
         b_ih[2 * H:3 * H]]).reshape(1, GW)
    bhh_row = jnp.concatenate(
        [jnp.zeros((2 * H,), jnp.float32), b_hh[2 * H:3 * H]]).reshape(1, GW)
    wih_aug = jnp.concatenate([wih, bih_row, zr], axis=0).astype(jnp.bfloat16)
    whh_aug = jnp.concatenate([whh, bhh_row, zr], axis=0).astype(jnp.bfloat16)
    return wih_aug, whh_aug


@functools.partial(jax.jit, static_argnames=())
def kernel(x, edge_index, W_gcn, b_gcn, W_ih, W_hh, b_ih, b_hh):
    loops = jnp.arange(N, dtype=edge_index.dtype)
    epad = jnp.zeros((2, E_PAD - N_VALID), jnp.int32)
    ed2 = jnp.concatenate(
        [edge_index.astype(jnp.int32), jnp.stack([loops, loops]), epad],
        axis=1)
    ed = jnp.concatenate([ed2, jnp.zeros((6, E_PAD), jnp.int32)], axis=0)

    xt = x.reshape(B, T, N * F)
    bg_aug = jnp.concatenate(
        [jnp.tile(b_gcn, N), jnp.full((1,), 20.0, jnp.float32),
         jnp.zeros((HA - H - 1,), jnp.float32)]).reshape(1, HA)
    wih_aug, whh_aug = _prep_gate_weights(W_ih, W_hh, b_ih, b_hh)

    gout, h_last = pl.pallas_call(
        _fused_kernel,
        grid=(NC,),
        in_specs=[
            pl.BlockSpec((8, E_PAD), lambda c: (0, 0)),
            pl.BlockSpec((B, C, N * F), lambda c: (0, c, 0)),
            pl.BlockSpec((F, HG), lambda c: (0, 0)),
            pl.BlockSpec((1, HA), lambda c: (0, 0)),
            pl.BlockSpec((HA, GW), lambda c: (0, 0)),
            pl.BlockSpec((HA, GW), lambda c: (0, 0)),
        ],
        out_specs=[
            pl.BlockSpec((B, C, H), lambda c: (0, c, 0)),
            pl.BlockSpec((B, H), lambda c: (0, 0)),
        ],
        out_shape=[
            jax.ShapeDtypeStruct((B, T, H), jnp.float32),
            jax.ShapeDtypeStruct((B, H), jnp.float32),
        ],
        scratch_shapes=[
            pltpu.VMEM((N * F, HA), jnp.bfloat16),
            pltpu.VMEM((B, HA), jnp.float32),
            pltpu.VMEM((R, R), jnp.bfloat16),
        ],
        compiler_params=pltpu.CompilerParams(
            dimension_semantics=("arbitrary",),
        ),
    )(ed, xt, W_gcn, bg_aug, wih_aug, whh_aug)

    gcn_features = gout.reshape(B, T, N, HG)
    return gcn_features, h_last


# speedup vs baseline: 1.1910x; 1.1491x over previous
import functools

import jax
import jax.numpy as jnp
from jax import lax
from jax.experimental import pallas as pl
from jax.experimental.pallas import tpu as pltpu

B = 64
T = 72
N = 5
F = 10
HG = 64
H = 320
HA = 384
GP = 384
GW = 3 * GP
E_PAD = 32
N_VALID = 21
C = 8
NC = T // C
R = B * C


def _fused_kernel(ed_ref, xt_ref, wg_ref, bg_ref, wih_ref, whh_ref,
                  gout_ref, hout_ref, k_scr, h_scr, p_scr):
    c = pl.program_id(0)

    @pl.when(c == 0)
    def _init():
        s_row = ed_ref[0:1, :]
        d_row = ed_ref[1:2, :]
        n_iota = lax.broadcasted_iota(jnp.int32, (8, E_PAD), 0)
        e_iota = lax.broadcasted_iota(jnp.int32, (8, E_PAD), 1)
        valid = (e_iota < N_VALID).astype(jnp.float32)
        oh_s = (jnp.broadcast_to(s_row, (8, E_PAD)) == n_iota)
        oh_d = (jnp.broadcast_to(d_row, (8, E_PAD)) == n_iota)
        oh_s = oh_s.astype(jnp.float32) * valid
        oh_d = oh_d.astype(jnp.float32) * valid
        deg = jnp.sum(oh_d, axis=1, keepdims=True)
        dis = jnp.where(deg > 0, lax.rsqrt(deg), 0.0)
        dis_s = jnp.sum(oh_s * dis, axis=0, keepdims=True)
        dis_d = jnp.sum(oh_d * dis, axis=0, keepdims=True)
        norm = dis_s * dis_d
        a8 = lax.dot_general(oh_d * norm, oh_s, (((1,), (1,)), ((), ())),
                             preferred_element_type=jnp.float32)

        r_i = lax.broadcasted_iota(jnp.int32, (N * F, 8), 0)
        c8_i = lax.broadcasted_iota(jnp.int32, (N * F, 8), 1)
        e_r = ((r_i // F) == c8_i).astype(jnp.float32)
        a_sel = lax.dot_general(e_r, a8, (((1,), (1,)), ((), ())),
                                preferred_element_type=jnp.float32)
        n8_i = lax.broadcasted_iota(jnp.int32, (8, H), 0)
        cH_i = lax.broadcasted_iota(jnp.int32, (8, H), 1)
        e_c = ((cH_i // HG) == n8_i).astype(jnp.float32)
        a_exp = jnp.dot(a_sel, e_c,
                        preferred_element_type=jnp.float32)
        rf_i = lax.broadcasted_iota(jnp.int32, (N * F, F), 0)
        cf_i = lax.broadcasted_iota(jnp.int32, (N * F, F), 1)
        f_r = ((rf_i % F) == cf_i).astype(jnp.float32)
        w_mid = jnp.dot(f_r, wg_ref[:],
                        preferred_element_type=jnp.float32)
        h_i = lax.broadcasted_iota(jnp.int32, (HG, H), 0)
        ch_i = lax.broadcasted_iota(jnp.int32, (HG, H), 1)
        f_c = ((ch_i % HG) == h_i).astype(jnp.float32)
        w_exp = jnp.dot(w_mid, f_c,
                        preferred_element_type=jnp.float32)
        kc_i = lax.broadcasted_iota(jnp.int32, (N * F, HA - H), 1)
        k_scr[:] = (jnp.concatenate(
            [a_exp * w_exp, jnp.zeros_like(kc_i, jnp.float32)],
            axis=1)).astype(jnp.bfloat16)

        hc_i = lax.broadcasted_iota(jnp.int32, (B, HA), 1)
        h_scr[:] = (hc_i == H).astype(jnp.float32)

        pr_i = lax.broadcasted_iota(jnp.int32, (R, R), 0)
        pc_i = lax.broadcasted_iota(jnp.int32, (R, R), 1)
        p_scr[:] = ((pr_i % B) * C + pr_i // B == pc_i).astype(jnp.bfloat16)

    xt = xt_ref[:].reshape(R, N * F)
    g = jnp.tanh(jnp.dot(xt.astype(jnp.bfloat16), k_scr[:],
                         preferred_element_type=jnp.float32)
                 + bg_ref[:])
    gout_ref[:] = g[:, 0:H].reshape(B, C, H)

    g_t = jnp.dot(p_scr[:], g.astype(jnp.bfloat16),
                  preferred_element_type=jnp.float32
                  ).astype(jnp.bfloat16)

    gi = jnp.dot(g_t, wih_ref[:],
                 preferred_element_type=jnp.float32)

    for i in range(C):
        gi_i = gi[i * B:(i + 1) * B, :]
        h_aug = h_scr[:]
        gh = jnp.dot(h_aug.astype(jnp.bfloat16), whh_ref[:],
                     preferred_element_type=jnp.float32)
        t_rz = jnp.tanh(gi_i[:, 0:GP + H] + gh[:, 0:GP + H])
        r = 0.5 + 0.5 * t_rz[:, 0:H]
        z = 0.5 + 0.5 * t_rz[:, GP:GP + H]
        n = jnp.tanh(gi_i[:, 2 * GP:2 * GP + H] + r * gh[:, 2 * GP:2 * GP + H])
        h_new = n + z * (h_aug[:, 0:H] - n)
        h_scr[:, 0:H] = h_new
    hout_ref[:] = h_scr[:, 0:H]


def _prep_gate_weights(W_ih, W_hh, b_ih, b_hh):
    zw = jnp.zeros((H, GP - H), jnp.float32)
    zr = jnp.zeros((HA - H - 1, GW), jnp.float32)

    def gate(w, gidx):
        blk = w[gidx * H:(gidx + 1) * H].T
        return blk if gidx == 2 else 0.5 * blk

    wih = jnp.concatenate(
        [gate(W_ih, 0), zw, gate(W_ih, 1), zw, gate(W_ih, 2), zw], axis=1)
    whh = jnp.concatenate(
        [gate(W_hh, 0), zw, gate(W_hh, 1), zw, gate(W_hh, 2), zw], axis=1)
    zb = jnp.zeros((GP - H,), jnp.float32)
    bih_row = jnp.concatenate(
        [0.5 * (b_ih[0:H] + b_hh[0:H]), zb,
         0.5 * (b_ih[H:2 * H] + b_hh[H:2 * H]), zb,
         b_ih[2 * H:3 * H], zb]).reshape(1, GW)
    bhh_row = jnp.concatenate(
        [jnp.zeros((2 * GP,), jnp.float32), b_hh[2 * H:3 * H], zb]
    ).reshape(1, GW)
    wih_aug = jnp.concatenate([wih, bih_row, zr], axis=0).astype(jnp.bfloat16)
    whh_aug = jnp.concatenate([whh, bhh_row, zr], axis=0).astype(jnp.bfloat16)
    return wih_aug, whh_aug


@functools.partial(jax.jit, static_argnames=())
def kernel(x, edge_index, W_gcn, b_gcn, W_ih, W_hh, b_ih, b_hh):
    loops = jnp.arange(N, dtype=edge_index.dtype)
    epad = jnp.zeros((2, E_PAD - N_VALID), jnp.int32)
    ed2 = jnp.concatenate(
        [edge_index.astype(jnp.int32), jnp.stack([loops, loops]), epad],
        axis=1)
    ed = jnp.concatenate([ed2, jnp.zeros((6, E_PAD), jnp.int32)], axis=0)

    xt = x.reshape(B, T, N * F)
    bg_aug = jnp.concatenate(
        [jnp.tile(b_gcn, N), jnp.full((1,), 20.0, jnp.float32),
         jnp.zeros((HA - H - 1,), jnp.float32)]).reshape(1, HA)
    wih_aug, whh_aug = _prep_gate_weights(W_ih, W_hh, b_ih, b_hh)

    gout, h_last = pl.pallas_call(
        _fused_kernel,
        grid=(NC,),
        in_specs=[
            pl.BlockSpec((8, E_PAD), lambda c: (0, 0)),
            pl.BlockSpec((B, C, N * F), lambda c: (0, c, 0)),
            pl.BlockSpec((F, HG), lambda c: (0, 0)),
            pl.BlockSpec((1, HA), lambda c: (0, 0)),
            pl.BlockSpec((HA, GW), lambda c: (0, 0)),
            pl.BlockSpec((HA, GW), lambda c: (0, 0)),
        ],
        out_specs=[
            pl.BlockSpec((B, C, H), lambda c: (0, c, 0)),
            pl.BlockSpec((B, H), lambda c: (0, 0)),
        ],
        out_shape=[
            jax.ShapeDtypeStruct((B, T, H), jnp.float32),
            jax.ShapeDtypeStruct((B, H), jnp.float32),
        ],
        scratch_shapes=[
            pltpu.VMEM((N * F, HA), jnp.bfloat16),
            pltpu.VMEM((B, HA), jnp.float32),
            pltpu.VMEM((R, R), jnp.bfloat16),
        ],
        compiler_params=pltpu.CompilerParams(
            dimension_semantics=("arbitrary",),
        ),
    )(ed, xt, W_gcn, bg_aug, wih_aug, whh_aug)

    gcn_features = gout.reshape(B, T, N, HG)
    return gcn_features, h_last
